# baseline (speedup 1.0000x reference)
import functools

import jax
import jax.numpy as jnp
from jax import lax
from jax.experimental import pallas as pl
from jax.experimental.pallas import tpu as pltpu
from jax.experimental.pallas import tpu_sc as plsc

B = 4096
L = 200
D = 64
CTX = 200
V = 1000000
N = B * L

NC = 2
NS = 16
NW = NC * NS
R = N // NW
C = 128
SUB = 128
NCHUNK = R // C
NBUF = 4
IDX_ROWS = N // SUB
LANES = 16


def _impl(x2d, pos2d, embp, ptab):
    mesh = plsc.VectorSubcoreMesh(core_axis_name="c", subcore_axis_name="s")

    @functools.partial(
        pl.kernel,
        mesh=mesh,
        compiler_params=pltpu.CompilerParams(
            use_tc_tiling_on_sc=False, needs_layout_passes=False),
        out_type=jax.ShapeDtypeStruct((N, D), jnp.float32),
        scratch_types=[
            pltpu.VMEM((NBUF, 1, SUB), jnp.int32),
            pltpu.VMEM((NBUF, 1, SUB), jnp.int32),
            pltpu.VMEM((NBUF, 1, SUB), jnp.int32),
            pltpu.VMEM((NBUF, 1, SUB), jnp.int32),
            pltpu.VMEM((NBUF, C, 2 * D), jnp.float32),
            pltpu.VMEM((NBUF, C, D), jnp.float32),
            pltpu.VMEM_SHARED((CTX, D), jnp.float32),
            pltpu.SemaphoreType.DMA,
            pltpu.SemaphoreType.DMA,
            pltpu.SemaphoreType.DMA,
            pltpu.SemaphoreType.DMA,
        ],
    )
    def k(x_hbm, p_hbm, embp_hbm, ptab_hbm, out_hbm,
          idx_v, pidx_v, kidx_v, hoff_v, rows2_v, rows_v, ptab_s,
          sem_i, sem_g, sem_a, sem_o):
        wid = lax.axis_index("s") * NC + lax.axis_index("c")
        irow0 = wid * (R // SUB)
        row0 = wid * R
        iota = lax.iota(jnp.int32, LANES)

        def prep(b):
            for q in range(SUB // LANES):
                sl = pl.ds(q * LANES, LANES)
                v = idx_v[b, 0, sl]
                kidx_v[b, 0, sl] = lax.shift_right_logical(v, 1)
                hoff_v[b, 0, sl] = lax.shift_left(lax.bitwise_and(v, 1), 6)

        def compact(b):
            def col(c2, carry):
                for q in range(SUB // LANES):
                    sl = pl.ds(q * LANES, LANES)
                    rowv = iota + (q * LANES)
                    colv = hoff_v[b, 0, sl] + c2
                    val = plsc.load_gather(rows2_v.at[b], [rowv, colv])
                    plsc.store_scatter(
                        rows_v.at[b], [rowv, jnp.zeros((LANES,), jnp.int32) + c2],
                        val)
                return carry
            lax.fori_loop(0, D, col, 0)

        def issue_I(c, b):
            irow = irow0 + c
            pltpu.async_copy(x_hbm.at[pl.ds(irow, 1)], idx_v.at[b], sem_i)
            pltpu.async_copy(p_hbm.at[pl.ds(irow, 1)], pidx_v.at[b], sem_i)

        def wait_I(b):
            pltpu.make_async_copy(x_hbm.at[pl.ds(0, 1)], idx_v.at[b], sem_i).wait()
            pltpu.make_async_copy(p_hbm.at[pl.ds(0, 1)], pidx_v.at[b], sem_i).wait()

        def issue_E(b):
            pltpu.async_copy(embp_hbm.at[kidx_v.at[b, 0]], rows2_v.at[b], sem_g)

        def wait_E(b):
            pltpu.make_async_copy(
                embp_hbm.at[pl.ds(0, C)], rows2_v.at[b], sem_g).wait()

        def issue_P(b):
            pltpu.async_copy(
                ptab_s.at[pidx_v.at[b, 0]], rows_v.at[b], sem_a, add=True)

        def wait_P(b):
            pltpu.make_async_copy(
                out_hbm.at[pl.ds(0, C)], rows_v.at[b], sem_a).wait()

        def issue_S(c, b):
            pltpu.async_copy(rows_v.at[b], out_hbm.at[pl.ds(row0 + c * C, C)], sem_o)

        def wait_S(b):
            pltpu.make_async_copy(
                rows_v.at[b], out_hbm.at[pl.ds(0, C)], sem_o).wait()

        @pl.when(lax.axis_index("s") == 0)
        def _():
            pltpu.sync_copy(ptab_hbm, ptab_s)
        plsc.subcore_barrier()

        pltpu.sync_copy(x_hbm.at[pl.ds(irow0, 1)], idx_v.at[0])
        pltpu.sync_copy(p_hbm.at[pl.ds(irow0, 1)], pidx_v.at[0])
        prep(0)
        issue_E(0)
        issue_I(1, 1)
        for c in (1, 2, 3):
            b = c % NBUF
            wait_I(b)
            prep(b)
            issue_E(b)
            issue_I(c + 1, (c + 1) % NBUF)
            wait_E((c - 1) % NBUF)
            compact((c - 1) % NBUF)
            issue_P((c - 1) % NBUF)
            if c >= 2:
                wait_P((c - 2) % NBUF)
                issue_S(c - 2, (c - 2) % NBUF)

        def body(t, carry):
            for b in range(NBUF):
                c = t * NBUF + b
                wait_S(b)
                wait_I(b)
                prep(b)
                issue_E(b)
                @pl.when(c < NCHUNK - 1)
                def _():
                    issue_I(c + 1, (b + 1) % NBUF)
                wait_E((b - 1) % NBUF)
                compact((b - 1) % NBUF)
                issue_P((b - 1) % NBUF)
                wait_P((b - 2) % NBUF)
                issue_S(c - 2, (b - 2) % NBUF)
            return carry

        lax.fori_loop(1, NCHUNK // NBUF, body, 0)

        last = NCHUNK - 1
        wait_E(last % NBUF)
        compact(last % NBUF)
        issue_P(last % NBUF)
        wait_P((last - 1) % NBUF)
        issue_S(last - 1, (last - 1) % NBUF)
        wait_P(last % NBUF)
        issue_S(last, last % NBUF)
        for b in range(NBUF):
            wait_S(b)

    return k(x2d, pos2d, embp, ptab)


def kernel(x, _pos, embedding_table, positional_table):
    x2d = x.T.reshape(IDX_ROWS, SUB)
    pos2d = _pos.T.reshape(IDX_ROWS, SUB)
    embp = embedding_table.reshape(V // 2, 2 * D)
    out = _impl(x2d, pos2d, embp, positional_table)
    return out.reshape(L, B, D).transpose(1, 0, 2)


# speedup vs baseline: 3.6610x; 3.6610x over previous
import functools

import jax
import jax.numpy as jnp
from jax import lax
from jax.experimental import pallas as pl
from jax.experimental.pallas import tpu as pltpu
from jax.experimental.pallas import tpu_sc as plsc

B = 4096
L = 200
D = 64
CTX = 200
V = 1000000
N = B * L

NC = 2
NS = 16
NW = NC * NS
R = N // NW
C = 128
SUB = 128
NSUB = C // SUB
NCHUNK = R // C
W = 2 * D
NBUF = 4
IDX_ROWS = N // SUB


def _impl(x2d, pos2d, emb, ptab):
    mesh = plsc.VectorSubcoreMesh(core_axis_name="c", subcore_axis_name="s")

    @functools.partial(
        pl.kernel,
        mesh=mesh,
        compiler_params=pltpu.CompilerParams(use_tc_tiling_on_sc=False),
        out_type=jax.ShapeDtypeStruct((N, W), jnp.float32),
        scratch_types=[
            pltpu.VMEM((NBUF, NSUB, SUB), jnp.int32),
            pltpu.VMEM((NBUF, NSUB, SUB), jnp.int32),
            pltpu.VMEM((NBUF, C, W), jnp.float32),
            pltpu.VMEM_SHARED((CTX, W), jnp.float32),
            pltpu.SemaphoreType.DMA,
            pltpu.SemaphoreType.DMA,
            pltpu.SemaphoreType.DMA,
            pltpu.SemaphoreType.DMA,
        ],
    )
    def k(x_hbm, p_hbm, emb_hbm, ptab_hbm, out_hbm,
          idx_v, pidx_v, rows_v, ptab_s, sem_i, sem_g, sem_a, sem_o):
        wid = lax.axis_index("s") * NC + lax.axis_index("c")
        irow0 = wid * (R // SUB)
        row0 = wid * R
        def issue_I(c, b):
            irow = irow0 + c * NSUB
            pltpu.async_copy(x_hbm.at[pl.ds(irow, NSUB)], idx_v.at[b], sem_i)
            pltpu.async_copy(p_hbm.at[pl.ds(irow, NSUB)], pidx_v.at[b], sem_i)

        def wait_I(b):
            pltpu.make_async_copy(x_hbm.at[pl.ds(0, NSUB)], idx_v.at[b], sem_i).wait()
            pltpu.make_async_copy(p_hbm.at[pl.ds(0, NSUB)], pidx_v.at[b], sem_i).wait()

        def issue_E(b):
            for j in range(NSUB):
                pltpu.async_copy(
                    emb_hbm.at[idx_v.at[b, j]],
                    rows_v.at[b].at[pl.ds(j * SUB, SUB)], sem_g)

        def wait_E(b):
            pltpu.make_async_copy(
                out_hbm.at[pl.ds(0, C)], rows_v.at[b], sem_g).wait()

        def issue_P(b):
            for j in range(NSUB):
                pltpu.async_copy(
                    ptab_s.at[pidx_v.at[b, j]],
                    rows_v.at[b].at[pl.ds(j * SUB, SUB)], sem_a, add=True)

        def wait_P(b):
            pltpu.make_async_copy(
                out_hbm.at[pl.ds(0, C)], rows_v.at[b], sem_a).wait()

        def issue_S(c, b):
            pltpu.async_copy(rows_v.at[b], out_hbm.at[pl.ds(row0 + c * C, C)], sem_o)

        def wait_S(b):
            pltpu.make_async_copy(
                rows_v.at[b], out_hbm.at[pl.ds(0, C)], sem_o).wait()

        @pl.when(lax.axis_index("s") == 0)
        def _():
            pltpu.sync_copy(ptab_hbm, ptab_s)
        plsc.subcore_barrier()

        pltpu.sync_copy(x_hbm.at[pl.ds(irow0, NSUB)], idx_v.at[0])
        pltpu.sync_copy(p_hbm.at[pl.ds(irow0, NSUB)], pidx_v.at[0])
        issue_E(0)
        issue_I(1, 1)
        for c in (1, 2, 3):
            b = c % NBUF
            wait_I(b)
            issue_E(b)
            issue_I(c + 1, (c + 1) % NBUF)
            wait_E((c - 1) % NBUF)
            issue_P((c - 1) % NBUF)
            if c >= 2:
                wait_P((c - 2) % NBUF)
                issue_S(c - 2, (c - 2) % NBUF)

        def body(t, carry):
            for b in range(NBUF):
                c = t * NBUF + b
                wait_S(b)
                wait_I(b)
                issue_E(b)
                @pl.when(c < NCHUNK - 1)
                def _():
                    issue_I(c + 1, (b + 1) % NBUF)
                wait_E((b - 1) % NBUF)
                issue_P((b - 1) % NBUF)
                wait_P((b - 2) % NBUF)
                issue_S(c - 2, (b - 2) % NBUF)
            return carry

        lax.fori_loop(1, NCHUNK // NBUF, body, 0)

        last = NCHUNK - 1
        wait_E(last % NBUF)
        issue_P(last % NBUF)
        wait_P((last - 1) % NBUF)
        issue_S(last - 1, (last - 1) % NBUF)
        wait_P(last % NBUF)
        issue_S(last, last % NBUF)
        for b in range(NBUF):
            wait_S(b)

    return k(x2d, pos2d, emb, ptab)


def kernel(x, _pos, embedding_table, positional_table):
    x2d = x.T.reshape(IDX_ROWS, SUB)
    pos2d = _pos.T.reshape(IDX_ROWS, SUB)
    embp = jnp.pad(embedding_table, ((0, 0), (0, D)))
    ptabp = jnp.pad(positional_table, ((0, 0), (0, D)))
    out = _impl(x2d, pos2d, embp, ptabp)
    return out[:, :D].reshape(L, B, D).transpose(1, 0, 2)
